# Initial kernel scaffold; baseline (speedup 1.0000x reference)
import jax
import jax.numpy as jnp
from jax.experimental import pallas as pl


def kernel(x, x_lengths, y, y_lengths, embed_en, W_ih_e, W_hh_e, b_ih_e, b_hh_e, embed_cn, W_ih_d, W_hh_d, b_ih_d, b_hh_d, fc_W, fc_b):
    raise NotImplementedError("write your pallas kernel here")


# speedup vs baseline: 5.8660x; 5.8660x over previous
import functools

import jax
import jax.numpy as jnp
from jax.experimental import pallas as pl
from jax.experimental.pallas import tpu as pltpu

B = 64
L = 512
H = 512
T = 32
NCHUNK = L // T


def _gru_steps(i, t0, n_steps, lengths, whh, bhh, xp_scr, h, o_scr):
    def step(t, h):
        hp = jnp.dot(h, whh, preferred_element_type=jnp.float32) + bhh
        xp_t = xp_scr[pl.ds(t * B, B), :]
        r = jax.nn.sigmoid(xp_t[:, :H] + hp[:, :H])
        z = jax.nn.sigmoid(xp_t[:, H:2 * H] + hp[:, H:2 * H])
        n = jnp.tanh(xp_t[:, 2 * H:] + r * hp[:, 2 * H:])
        h_new = (1.0 - z) * n + z * h
        m = (i * T + t) < lengths
        if o_scr is not None:
            o_scr[pl.ds(t * B, B), :] = jnp.where(m, h_new, 0.0)
        return jnp.where(m, h_new, h)

    return jax.lax.fori_loop(t0, t0 + n_steps, step, h)


def _enc_kernel(len_ref, emb_ref, wih_ref, whh_ref, bih_ref, bhh_ref,
                hout_ref, xp_scr, h_scr):
    i = pl.program_id(0)

    @pl.when(i == 0)
    def _():
        h_scr[...] = jnp.zeros((B, H), jnp.float32)

    xp_scr[...] = (
        jnp.dot(emb_ref[...], wih_ref[...], preferred_element_type=jnp.float32)
        + bih_ref[...])

    h = _gru_steps(i, 0, T, len_ref[...], whh_ref[...], bhh_ref[...],
                   xp_scr, h_scr[...], None)
    h_scr[...] = h
    hout_ref[...] = h


def _dec_kernel(len_ref, emb_ref, wih_ref, whh_ref, bih_ref, bhh_ref,
                h0_ref, fcw_ref, fcb_ref, out_ref, xp_scr, o_scr, h_scr):
    i = pl.program_id(0)

    @pl.when(i == 0)
    def _():
        h_scr[...] = h0_ref[...]

    xp_scr[...] = (
        jnp.dot(emb_ref[...], wih_ref[...], preferred_element_type=jnp.float32)
        + bih_ref[...])

    h = _gru_steps(i, 0, T, len_ref[...], whh_ref[...], bhh_ref[...],
                   xp_scr, h_scr[...], o_scr)
    h_scr[...] = h

    logits = (
        jnp.dot(o_scr[...], fcw_ref[...], preferred_element_type=jnp.float32)
        + fcb_ref[...])
    mx = jnp.max(logits, axis=-1, keepdims=True)
    lse = jnp.log(jnp.sum(jnp.exp(logits - mx), axis=-1, keepdims=True)) + mx
    out_ref[...] = logits - lse


def _const_spec(shape):
    return pl.BlockSpec(shape, lambda i: tuple(0 for _ in shape))


def _run_encoder(lengths, emb, wih_t, whh_t, bih, bhh, interpret=False):
    return pl.pallas_call(
        _enc_kernel,
        grid=(NCHUNK,),
        in_specs=[
            _const_spec((B, 1)),
            pl.BlockSpec((T * B, H), lambda i: (i, 0)),
            _const_spec((H, 3 * H)),
            _const_spec((H, 3 * H)),
            _const_spec((1, 3 * H)),
            _const_spec((1, 3 * H)),
        ],
        out_specs=_const_spec((B, H)),
        out_shape=jax.ShapeDtypeStruct((B, H), jnp.float32),
        scratch_shapes=[
            pltpu.VMEM((T * B, 3 * H), jnp.float32),
            pltpu.VMEM((B, H), jnp.float32),
        ],
        interpret=interpret,
    )(lengths, emb, wih_t, whh_t, bih, bhh)


def _run_decoder(lengths, emb, wih_t, whh_t, bih, bhh, h0, fcw_t, fcb,
                 interpret=False):
    return pl.pallas_call(
        _dec_kernel,
        grid=(NCHUNK,),
        in_specs=[
            _const_spec((B, 1)),
            pl.BlockSpec((T * B, H), lambda i: (i, 0)),
            _const_spec((H, 3 * H)),
            _const_spec((H, 3 * H)),
            _const_spec((1, 3 * H)),
            _const_spec((1, 3 * H)),
            _const_spec((B, H)),
            _const_spec((H, H)),
            _const_spec((1, H)),
        ],
        out_specs=pl.BlockSpec((T * B, H), lambda i: (i, 0)),
        out_shape=jax.ShapeDtypeStruct((L * B, H), jnp.float32),
        scratch_shapes=[
            pltpu.VMEM((T * B, 3 * H), jnp.float32),
            pltpu.VMEM((T * B, H), jnp.float32),
            pltpu.VMEM((B, H), jnp.float32),
        ],
        interpret=interpret,
    )(lengths, emb, wih_t, whh_t, bih, bhh, h0, fcw_t, fcb)


def kernel(x, x_lengths, y, y_lengths, embed_en, W_ih_e, W_hh_e, b_ih_e,
           b_hh_e, embed_cn, W_ih_d, W_hh_d, b_ih_d, b_hh_d, fc_W, fc_b,
           interpret=False):
    enc_emb = jnp.take(embed_en, x.T.reshape(-1), axis=0)
    dec_emb = jnp.take(embed_cn, y.T.reshape(-1), axis=0)

    xlen = x_lengths.astype(jnp.int32).reshape(B, 1)
    ylen = y_lengths.astype(jnp.int32).reshape(B, 1)

    enc_h = _run_encoder(xlen, enc_emb, W_ih_e.T, W_hh_e.T,
                         b_ih_e.reshape(1, -1), b_hh_e.reshape(1, -1),
                         interpret=interpret)
    out2d = _run_decoder(ylen, dec_emb, W_ih_d.T, W_hh_d.T,
                         b_ih_d.reshape(1, -1), b_hh_d.reshape(1, -1),
                         enc_h, fc_W.T, fc_b.reshape(1, -1),
                         interpret=interpret)
    return out2d.reshape(L, B, H).swapaxes(0, 1)


# speedup vs baseline: 5.9629x; 1.0165x over previous
import functools

import jax
import jax.numpy as jnp
from jax.experimental import pallas as pl
from jax.experimental.pallas import tpu as pltpu

B = 64
L = 512
H = 512
T = 32
NCHUNK = L // T


def _gru_steps(i, t0, n_steps, lengths, whh, bhh, xp_scr, h, o_scr):
    def step(t, h):
        hp = jnp.dot(h.astype(jnp.bfloat16), whh,
                     preferred_element_type=jnp.float32) + bhh
        xp_t = xp_scr[pl.ds(t * B, B), :]
        r = jax.nn.sigmoid(xp_t[:, :H] + hp[:, :H])
        z = jax.nn.sigmoid(xp_t[:, H:2 * H] + hp[:, H:2 * H])
        n = jnp.tanh(xp_t[:, 2 * H:] + r * hp[:, 2 * H:])
        h_new = (1.0 - z) * n + z * h
        m = (i * T + t) < lengths
        if o_scr is not None:
            o_scr[pl.ds(t * B, B), :] = jnp.where(m, h_new, 0.0)
        return jnp.where(m, h_new, h)

    return jax.lax.fori_loop(t0, t0 + n_steps, step, h)


def _enc_kernel(len_ref, emb_ref, wih_ref, whh_ref, bih_ref, bhh_ref,
                hout_ref, xp_scr, h_scr):
    i = pl.program_id(0)

    @pl.when(i == 0)
    def _():
        h_scr[...] = jnp.zeros((B, H), jnp.float32)

    xp_scr[...] = (
        jnp.dot(emb_ref[...].astype(jnp.bfloat16), wih_ref[...],
                preferred_element_type=jnp.float32)
        + bih_ref[...])

    h = _gru_steps(i, 0, T, len_ref[...], whh_ref[...], bhh_ref[...],
                   xp_scr, h_scr[...], None)
    h_scr[...] = h
    hout_ref[...] = h


def _dec_kernel(len_ref, emb_ref, wih_ref, whh_ref, bih_ref, bhh_ref,
                h0_ref, fcw_ref, fcb_ref, out_ref, xp_scr, o_scr, h_scr):
    i = pl.program_id(0)

    @pl.when(i == 0)
    def _():
        h_scr[...] = h0_ref[...]

    xp_scr[...] = (
        jnp.dot(emb_ref[...].astype(jnp.bfloat16), wih_ref[...],
                preferred_element_type=jnp.float32)
        + bih_ref[...])

    h = _gru_steps(i, 0, T, len_ref[...], whh_ref[...], bhh_ref[...],
                   xp_scr, h_scr[...], o_scr)
    h_scr[...] = h

    logits = (
        jnp.dot(o_scr[...].astype(jnp.bfloat16), fcw_ref[...],
                preferred_element_type=jnp.float32)
        + fcb_ref[...])
    mx = jnp.max(logits, axis=-1, keepdims=True)
    lse = jnp.log(jnp.sum(jnp.exp(logits - mx), axis=-1, keepdims=True)) + mx
    out_ref[...] = logits - lse


def _const_spec(shape):
    return pl.BlockSpec(shape, lambda i: tuple(0 for _ in shape))


def _run_encoder(lengths, emb, wih_t, whh_t, bih, bhh, interpret=False):
    return pl.pallas_call(
        _enc_kernel,
        grid=(NCHUNK,),
        in_specs=[
            _const_spec((B, 1)),
            pl.BlockSpec((T * B, H), lambda i: (i, 0)),
            _const_spec((H, 3 * H)),
            _const_spec((H, 3 * H)),
            _const_spec((1, 3 * H)),
            _const_spec((1, 3 * H)),
        ],
        out_specs=_const_spec((B, H)),
        out_shape=jax.ShapeDtypeStruct((B, H), jnp.float32),
        scratch_shapes=[
            pltpu.VMEM((T * B, 3 * H), jnp.float32),
            pltpu.VMEM((B, H), jnp.float32),
        ],
        interpret=interpret,
    )(lengths, emb, wih_t, whh_t, bih, bhh)


def _run_decoder(lengths, emb, wih_t, whh_t, bih, bhh, h0, fcw_t, fcb,
                 interpret=False):
    return pl.pallas_call(
        _dec_kernel,
        grid=(NCHUNK,),
        in_specs=[
            _const_spec((B, 1)),
            pl.BlockSpec((T * B, H), lambda i: (i, 0)),
            _const_spec((H, 3 * H)),
            _const_spec((H, 3 * H)),
            _const_spec((1, 3 * H)),
            _const_spec((1, 3 * H)),
            _const_spec((B, H)),
            _const_spec((H, H)),
            _const_spec((1, H)),
        ],
        out_specs=pl.BlockSpec((T * B, H), lambda i: (i, 0)),
        out_shape=jax.ShapeDtypeStruct((L * B, H), jnp.float32),
        scratch_shapes=[
            pltpu.VMEM((T * B, 3 * H), jnp.float32),
            pltpu.VMEM((T * B, H), jnp.float32),
            pltpu.VMEM((B, H), jnp.float32),
        ],
        interpret=interpret,
    )(lengths, emb, wih_t, whh_t, bih, bhh, h0, fcw_t, fcb)


def kernel(x, x_lengths, y, y_lengths, embed_en, W_ih_e, W_hh_e, b_ih_e,
           b_hh_e, embed_cn, W_ih_d, W_hh_d, b_ih_d, b_hh_d, fc_W, fc_b,
           interpret=False):
    enc_emb = jnp.take(embed_en, x.T.reshape(-1), axis=0)
    dec_emb = jnp.take(embed_cn, y.T.reshape(-1), axis=0)

    xlen = x_lengths.astype(jnp.int32).reshape(B, 1)
    ylen = y_lengths.astype(jnp.int32).reshape(B, 1)

    bf = jnp.bfloat16
    enc_h = _run_encoder(xlen, enc_emb, W_ih_e.T.astype(bf),
                         W_hh_e.T.astype(bf),
                         b_ih_e.reshape(1, -1), b_hh_e.reshape(1, -1),
                         interpret=interpret)
    out2d = _run_decoder(ylen, dec_emb, W_ih_d.T.astype(bf),
                         W_hh_d.T.astype(bf),
                         b_ih_d.reshape(1, -1), b_hh_d.reshape(1, -1),
                         enc_h, fc_W.T.astype(bf), fc_b.reshape(1, -1),
                         interpret=interpret)
    return out2d.reshape(L, B, H).swapaxes(0, 1)


# speedup vs baseline: 7.2802x; 1.2209x over previous
import functools

import jax
import jax.numpy as jnp
from jax.experimental import pallas as pl
from jax.experimental.pallas import tpu as pltpu

B = 64
L = 512
H = 512
T = 32
NCHUNK = L // T


def _gru_steps(i, t0, n_steps, lengths, whh, bhh, xp_scr, h, o_scr):
    def step(t, h):
        hp = jnp.dot(h.astype(jnp.bfloat16), whh,
                     preferred_element_type=jnp.float32) + bhh
        xp_t = xp_scr[pl.ds(t * B, B), :]
        r = jax.nn.sigmoid(xp_t[:, :H] + hp[:, :H])
        z = jax.nn.sigmoid(xp_t[:, H:2 * H] + hp[:, H:2 * H])
        n = jnp.tanh(xp_t[:, 2 * H:] + r * hp[:, 2 * H:])
        h_new = (1.0 - z) * n + z * h
        m = (i * T + t) < lengths
        if o_scr is not None:
            o_scr[pl.ds(t * B, B), :] = jnp.where(m, h_new, 0.0)
        return jnp.where(m, h_new, h)

    return jax.lax.fori_loop(t0, t0 + n_steps, step, h)


def _enc_kernel(len_ref, emb_ref, wih_ref, whh_ref, bih_ref, bhh_ref,
                hout_ref, xp_scr, h_scr):
    i = pl.program_id(0)

    @pl.when(i == 0)
    def _():
        h_scr[...] = jnp.zeros((B, H), jnp.float32)

    xp_scr[...] = (
        jnp.dot(emb_ref[...].astype(jnp.bfloat16), wih_ref[...],
                preferred_element_type=jnp.float32)
        + bih_ref[...])

    h = _gru_steps(i, 0, T, len_ref[...], whh_ref[...], bhh_ref[...],
                   xp_scr, h_scr[...], None)
    h_scr[...] = h
    hout_ref[...] = h


def _dec_kernel(len_ref, emb_ref, wih_ref, whh_ref, bih_ref, bhh_ref,
                h0_ref, fcw_ref, fcb_ref, out_ref, xp_scr, o_scr, h_scr):
    i = pl.program_id(0)

    @pl.when(i == 0)
    def _():
        h_scr[...] = h0_ref[...]

    xp_scr[...] = (
        jnp.dot(emb_ref[...].astype(jnp.bfloat16), wih_ref[...],
                preferred_element_type=jnp.float32)
        + bih_ref[...])

    h = _gru_steps(i, 0, T, len_ref[...], whh_ref[...], bhh_ref[...],
                   xp_scr, h_scr[...], o_scr)
    h_scr[...] = h

    logits = (
        jnp.dot(o_scr[...].astype(jnp.bfloat16), fcw_ref[...],
                preferred_element_type=jnp.float32)
        + fcb_ref[...])
    mx = jnp.max(logits, axis=-1, keepdims=True)
    lse = jnp.log(jnp.sum(jnp.exp(logits - mx), axis=-1, keepdims=True)) + mx
    out = logits - lse
    out_ref[...] = jnp.swapaxes(out.reshape(T, B, H), 0, 1)


def _const_spec(shape):
    return pl.BlockSpec(shape, lambda i: tuple(0 for _ in shape))


def _run_encoder(lengths, emb, wih_t, whh_t, bih, bhh, interpret=False):
    return pl.pallas_call(
        _enc_kernel,
        grid=(NCHUNK,),
        in_specs=[
            _const_spec((B, 1)),
            pl.BlockSpec((T * B, H), lambda i: (i, 0)),
            _const_spec((H, 3 * H)),
            _const_spec((H, 3 * H)),
            _const_spec((1, 3 * H)),
            _const_spec((1, 3 * H)),
        ],
        out_specs=_const_spec((B, H)),
        out_shape=jax.ShapeDtypeStruct((B, H), jnp.float32),
        scratch_shapes=[
            pltpu.VMEM((T * B, 3 * H), jnp.float32),
            pltpu.VMEM((B, H), jnp.float32),
        ],
        interpret=interpret,
    )(lengths, emb, wih_t, whh_t, bih, bhh)


def _run_decoder(lengths, emb, wih_t, whh_t, bih, bhh, h0, fcw_t, fcb,
                 interpret=False):
    return pl.pallas_call(
        _dec_kernel,
        grid=(NCHUNK,),
        in_specs=[
            _const_spec((B, 1)),
            pl.BlockSpec((T * B, H), lambda i: (i, 0)),
            _const_spec((H, 3 * H)),
            _const_spec((H, 3 * H)),
            _const_spec((1, 3 * H)),
            _const_spec((1, 3 * H)),
            _const_spec((B, H)),
            _const_spec((H, H)),
            _const_spec((1, H)),
        ],
        out_specs=pl.BlockSpec((B, T, H), lambda i: (0, i, 0)),
        out_shape=jax.ShapeDtypeStruct((B, L, H), jnp.float32),
        scratch_shapes=[
            pltpu.VMEM((T * B, 3 * H), jnp.float32),
            pltpu.VMEM((T * B, H), jnp.float32),
            pltpu.VMEM((B, H), jnp.float32),
        ],
        interpret=interpret,
    )(lengths, emb, wih_t, whh_t, bih, bhh, h0, fcw_t, fcb)


def kernel(x, x_lengths, y, y_lengths, embed_en, W_ih_e, W_hh_e, b_ih_e,
           b_hh_e, embed_cn, W_ih_d, W_hh_d, b_ih_d, b_hh_d, fc_W, fc_b,
           interpret=False):
    enc_emb = embed_en.at[x.T.reshape(-1)].get(
        mode='promise_in_bounds')
    dec_emb = embed_cn.at[y.T.reshape(-1)].get(
        mode='promise_in_bounds')

    xlen = x_lengths.astype(jnp.int32).reshape(B, 1)
    ylen = y_lengths.astype(jnp.int32).reshape(B, 1)

    bf = jnp.bfloat16
    enc_h = _run_encoder(xlen, enc_emb, W_ih_e.T.astype(bf),
                         W_hh_e.T.astype(bf),
                         b_ih_e.reshape(1, -1), b_hh_e.reshape(1, -1),
                         interpret=interpret)
    out2d = _run_decoder(ylen, dec_emb, W_ih_d.T.astype(bf),
                         W_hh_d.T.astype(bf),
                         b_ih_d.reshape(1, -1), b_hh_d.reshape(1, -1),
                         enc_h, fc_W.T.astype(bf), fc_b.reshape(1, -1),
                         interpret=interpret)
    return out2d


# speedup vs baseline: 7.4759x; 1.0269x over previous
import functools

import jax
import jax.numpy as jnp
from jax.experimental import pallas as pl
from jax.experimental.pallas import tpu as pltpu

B = 64
L = 512
H = 512
T = 32
NCHUNK = L // T


def _gru_steps(i, t0, n_steps, lengths, whh, bhh, xp_scr, h, o_scr):
    def step(t, h):
        hp = jnp.dot(h.astype(jnp.bfloat16), whh,
                     preferred_element_type=jnp.float32) + bhh
        xp_t = xp_scr[pl.ds(t * B, B), :]
        g = (xp_t[:, :2 * H] + hp[:, :2 * H]).astype(jnp.bfloat16)
        r = jax.nn.sigmoid(g[:, :H])
        z = jax.nn.sigmoid(g[:, H:])
        n = jnp.tanh((xp_t[:, 2 * H:]
                      + r.astype(jnp.float32) * hp[:, 2 * H:]
                      ).astype(jnp.bfloat16)).astype(jnp.float32)
        zf = z.astype(jnp.float32)
        h_new = n + zf * (h - n)
        m = (i * T + t) < lengths
        if o_scr is not None:
            o_scr[pl.ds(t * B, B), :] = jnp.where(m, h_new, 0.0)
        return jnp.where(m, h_new, h)

    return jax.lax.fori_loop(t0, t0 + n_steps, step, h, unroll=2)


def _enc_kernel(len_ref, emb_ref, wih_ref, whh_ref, bih_ref, bhh_ref,
                hout_ref, xp_scr, h_scr):
    i = pl.program_id(0)

    @pl.when(i == 0)
    def _():
        h_scr[...] = jnp.zeros((B, H), jnp.float32)

    xp_scr[...] = (
        jnp.dot(emb_ref[...].astype(jnp.bfloat16), wih_ref[...],
                preferred_element_type=jnp.float32)
        + bih_ref[...])

    h = _gru_steps(i, 0, T, len_ref[...], whh_ref[...], bhh_ref[...],
                   xp_scr, h_scr[...], None)
    h_scr[...] = h
    hout_ref[...] = h


def _dec_kernel(len_ref, emb_ref, wih_ref, whh_ref, bih_ref, bhh_ref,
                h0_ref, fcw_ref, fcb_ref, out_ref, xp_scr, o_scr, h_scr):
    i = pl.program_id(0)

    @pl.when(i == 0)
    def _():
        h_scr[...] = h0_ref[...]

    xp_scr[...] = (
        jnp.dot(emb_ref[...].astype(jnp.bfloat16), wih_ref[...],
                preferred_element_type=jnp.float32)
        + bih_ref[...])

    h = _gru_steps(i, 0, T, len_ref[...], whh_ref[...], bhh_ref[...],
                   xp_scr, h_scr[...], o_scr)
    h_scr[...] = h

    logits = (
        jnp.dot(o_scr[...].astype(jnp.bfloat16), fcw_ref[...],
                preferred_element_type=jnp.float32)
        + fcb_ref[...])
    mx = jnp.max(logits, axis=-1, keepdims=True)
    lse = jnp.log(jnp.sum(jnp.exp(logits - mx), axis=-1, keepdims=True)) + mx
    out = logits - lse
    out_ref[...] = jnp.swapaxes(out.reshape(T, B, H), 0, 1)


def _const_spec(shape):
    return pl.BlockSpec(shape, lambda i: tuple(0 for _ in shape))


def _run_encoder(lengths, emb, wih_t, whh_t, bih, bhh, interpret=False):
    return pl.pallas_call(
        _enc_kernel,
        grid=(NCHUNK,),
        in_specs=[
            _const_spec((B, 1)),
            pl.BlockSpec((T * B, H), lambda i: (i, 0)),
            _const_spec((H, 3 * H)),
            _const_spec((H, 3 * H)),
            _const_spec((1, 3 * H)),
            _const_spec((1, 3 * H)),
        ],
        out_specs=_const_spec((B, H)),
        out_shape=jax.ShapeDtypeStruct((B, H), jnp.float32),
        scratch_shapes=[
            pltpu.VMEM((T * B, 3 * H), jnp.float32),
            pltpu.VMEM((B, H), jnp.float32),
        ],
        interpret=interpret,
    )(lengths, emb, wih_t, whh_t, bih, bhh)


def _run_decoder(lengths, emb, wih_t, whh_t, bih, bhh, h0, fcw_t, fcb,
                 interpret=False):
    return pl.pallas_call(
        _dec_kernel,
        grid=(NCHUNK,),
        in_specs=[
            _const_spec((B, 1)),
            pl.BlockSpec((T * B, H), lambda i: (i, 0)),
            _const_spec((H, 3 * H)),
            _const_spec((H, 3 * H)),
            _const_spec((1, 3 * H)),
            _const_spec((1, 3 * H)),
            _const_spec((B, H)),
            _const_spec((H, H)),
            _const_spec((1, H)),
        ],
        out_specs=pl.BlockSpec((B, T, H), lambda i: (0, i, 0)),
        out_shape=jax.ShapeDtypeStruct((B, L, H), jnp.float32),
        scratch_shapes=[
            pltpu.VMEM((T * B, 3 * H), jnp.float32),
            pltpu.VMEM((T * B, H), jnp.float32),
            pltpu.VMEM((B, H), jnp.float32),
        ],
        interpret=interpret,
    )(lengths, emb, wih_t, whh_t, bih, bhh, h0, fcw_t, fcb)


def kernel(x, x_lengths, y, y_lengths, embed_en, W_ih_e, W_hh_e, b_ih_e,
           b_hh_e, embed_cn, W_ih_d, W_hh_d, b_ih_d, b_hh_d, fc_W, fc_b,
           interpret=False):
    enc_emb = embed_en.at[x.T.reshape(-1)].get(
        mode='promise_in_bounds')
    dec_emb = embed_cn.at[y.T.reshape(-1)].get(
        mode='promise_in_bounds')

    xlen = x_lengths.astype(jnp.int32).reshape(B, 1)
    ylen = y_lengths.astype(jnp.int32).reshape(B, 1)

    bf = jnp.bfloat16
    enc_h = _run_encoder(xlen, enc_emb, W_ih_e.T.astype(bf),
                         W_hh_e.T.astype(bf),
                         b_ih_e.reshape(1, -1), b_hh_e.reshape(1, -1),
                         interpret=interpret)
    out2d = _run_decoder(ylen, dec_emb, W_ih_d.T.astype(bf),
                         W_hh_d.T.astype(bf),
                         b_ih_d.reshape(1, -1), b_hh_d.reshape(1, -1),
                         enc_h, fc_W.T.astype(bf), fc_b.reshape(1, -1),
                         interpret=interpret)
    return out2d
